# baseline (speedup 1.0000x reference)
import functools

import jax
import jax.numpy as jnp
from jax import lax
from jax.experimental import pallas as pl
from jax.experimental.pallas import tpu as pltpu
from jax.experimental.pallas import tpu_sc as plsc

B = 1024
L = 512
D = 128
NC = 2
NS = 16
NW = NC * NS
BSC = 768
BPW = BSC // NW
ZR = 128


def _pos_body(table_hbm, len_hbm, out_hbm, lens_v, tbuf, zbuf, bbuf, sem, dsem):
    wid = lax.axis_index("s") * NC + lax.axis_index("c")
    pltpu.sync_copy(len_hbm.at[pl.ds(wid * BPW, BPW)], lens_v)
    tld = pltpu.async_copy(table_hbm, tbuf, dsem)

    zero = jnp.zeros((16,), jnp.float32)

    def zrow(r, carry):
        for j in range(D // 16):
            zbuf[r, pl.ds(j * 16, 16)] = zero
        return carry

    lax.fori_loop(0, ZR, zrow, 0)
    tld.wait()

    lane = lax.iota(jnp.int32, 16)
    lens_lo = lens_v[pl.ds(0, 16)]
    lens_hi = lens_v[pl.ds(BPW - 16, 16)]
    obase = wid * (BPW * L)

    def batch_body(b_loc, carry):
        len_s = jnp.maximum(
            jnp.max(jnp.where(lane == b_loc, lens_lo, 0)),
            jnp.max(jnp.where(lane + (BPW - 16) == b_loc, lens_hi, 0)))
        row0 = obase + b_loc * L
        q8 = len_s & ~7
        r8 = len_s & 7

        off = jnp.int32(0)
        for s in (256, 128, 64, 32, 16, 8):
            bit = q8 & s

            @pl.when(bit != 0)
            def _():
                pltpu.async_copy(
                    tbuf.at[pl.ds(pl.multiple_of(off, 8), s)],
                    out_hbm.at[pl.ds(pl.multiple_of(row0 + off, 8), s)], sem)

            off = off + bit

        bb = b_loc * 8
        for j in range(8):
            keep = j < r8
            for k in range(D // 16):
                tv = tbuf[q8 + j, pl.ds(k * 16, 16)]
                bbuf[bb + j, pl.ds(k * 16, 16)] = jnp.where(keep, tv, zero)
        pltpu.async_copy(
            bbuf.at[pl.ds(bb, 8)],
            out_hbm.at[pl.ds(pl.multiple_of(row0 + q8, 8), 8)], sem)
        off = off + 8

        t = L - q8 - 8
        for _i in range(3):
            c = t >= ZR

            @pl.when(c)
            def _():
                pltpu.async_copy(
                    zbuf, out_hbm.at[pl.ds(pl.multiple_of(row0 + off, 8), ZR)],
                    sem)

            dec = jnp.where(c, ZR, 0).astype(jnp.int32)
            off = off + dec
            t = t - dec
        for s in (64, 32, 16, 8):
            bit = t & s

            @pl.when(bit != 0)
            def _():
                pltpu.async_copy(
                    zbuf.at[pl.ds(0, s)],
                    out_hbm.at[pl.ds(pl.multiple_of(row0 + off, 8), s)], sem)

            off = off + bit
        return carry

    lax.fori_loop(0, BPW, batch_body, 0)

    def drain(i, carry):
        pltpu.make_async_copy(table_hbm, tbuf, sem).wait()
        return carry

    lax.fori_loop(0, BPW, drain, 0)


def _sc_part(table512, lens_sc):
    mesh = plsc.VectorSubcoreMesh(core_axis_name="c", subcore_axis_name="s")
    kfn = pl.kernel(
        _pos_body,
        out_type=jax.ShapeDtypeStruct((BSC * L, D), jnp.float32),
        mesh=mesh,
        scratch_types=[
            pltpu.VMEM((BPW,), jnp.int32),
            pltpu.VMEM((L, D), jnp.float32),
            pltpu.VMEM((ZR, D), jnp.float32),
            pltpu.VMEM((BPW * 8, D), jnp.float32),
            pltpu.SemaphoreType.DMA,
            pltpu.SemaphoreType.DMA,
        ],
        compiler_params=pltpu.CompilerParams(needs_layout_passes=False),
    )
    return kfn(table512, lens_sc).reshape(BSC, L, D)


def _tc_body(len_ref, table_ref, out_ref):
    b = pl.program_id(0)
    ln = len_ref[b]
    pos = lax.broadcasted_iota(jnp.int32, (L, D), 0) + 1
    out_ref[0] = jnp.where(pos <= ln, table_ref[...], 0.0)


def _tc_part(table512, lens_tc):
    nb = B - BSC
    return pl.pallas_call(
        _tc_body,
        grid=(nb,),
        in_specs=[
            pl.BlockSpec(memory_space=pltpu.SMEM),
            pl.BlockSpec((L, D), lambda b: (0, 0)),
        ],
        out_specs=pl.BlockSpec((1, L, D), lambda b: (b, 0, 0)),
        out_shape=jax.ShapeDtypeStruct((nb, L, D), jnp.float32),
    )(lens_tc, table512)


@functools.partial(jax.jit)
def kernel(pos_enc, input_len):
    table512 = pos_enc[1:L + 1]
    ilen = input_len.astype(jnp.int32)
    out_sc = _sc_part(table512, ilen[:BSC])
    out_tc = _tc_part(table512, ilen[BSC:])
    return jnp.concatenate([out_sc, out_tc], axis=0)


# speedup vs baseline: 2.7765x; 2.7765x over previous
import functools

import jax
import jax.numpy as jnp
from jax import lax
from jax.experimental import pallas as pl
from jax.experimental.pallas import tpu as pltpu
from jax.experimental.pallas import tpu_sc as plsc

B = 1024
L = 512
D = 128
NC = 2
NS = 16
NW = NC * NS
BPW = B // NW
ZR = 128


def _pos_body(table_hbm, len_hbm, out_hbm, lens_v, tbuf, zbuf, bbuf, sem, dsem):
    wid = lax.axis_index("s") * NC + lax.axis_index("c")
    pltpu.sync_copy(len_hbm.at[pl.ds(wid * BPW, BPW)], lens_v)
    tld = pltpu.async_copy(table_hbm.at[pl.ds(0, L + 16)], tbuf, dsem)

    zero = jnp.zeros((16,), jnp.float32)

    def zrow(r, carry):
        for j in range(D // 16):
            zbuf[r, pl.ds(j * 16, 16)] = zero
        return carry

    lax.fori_loop(0, ZR, zrow, 0)
    tld.wait()

    lane = lax.iota(jnp.int32, 16)
    lens_lo = lens_v[pl.ds(0, 16)]
    lens_hi = lens_v[pl.ds(16, 16)]
    obase = wid * (BPW * L)

    def batch_body(b_loc, carry):
        len_s = jnp.maximum(
            jnp.max(jnp.where(lane == b_loc, lens_lo, 0)),
            jnp.max(jnp.where(lane + 16 == b_loc, lens_hi, 0)))
        row0 = obase + b_loc * L
        q8 = len_s & ~7
        r8 = len_s & 7

        off = jnp.int32(0)
        for s in (256, 128, 64, 32, 16, 8):
            bit = q8 & s

            @pl.when(bit != 0)
            def _():
                pltpu.async_copy(
                    tbuf.at[pl.ds(off + 1, s)],
                    out_hbm.at[pl.ds(pl.multiple_of(row0 + off, 8), s)], sem)

            off = off + bit

        bb = b_loc * 8
        for j in range(8):
            keep = j < r8
            for k in range(D // 16):
                tv = tbuf[q8 + j + 1, pl.ds(k * 16, 16)]
                bbuf[bb + j, pl.ds(k * 16, 16)] = jnp.where(keep, tv, zero)
        pltpu.async_copy(
            bbuf.at[pl.ds(bb, 8)], out_hbm.at[pl.ds(pl.multiple_of(row0 + q8, 8), 8)], sem)
        off = off + 8

        t = L - q8 - 8
        for _i in range(3):
            c = t >= ZR

            @pl.when(c)
            def _():
                pltpu.async_copy(
                    zbuf, out_hbm.at[pl.ds(pl.multiple_of(row0 + off, 8), ZR)], sem)

            dec = jnp.where(c, ZR, 0).astype(jnp.int32)
            off = off + dec
            t = t - dec
        for s in (64, 32, 16, 8):
            bit = t & s

            @pl.when(bit != 0)
            def _():
                pltpu.async_copy(
                    zbuf.at[pl.ds(0, s)],
                    out_hbm.at[pl.ds(pl.multiple_of(row0 + off, 8), s)], sem)

            off = off + bit
        return carry

    lax.fori_loop(0, BPW, batch_body, 0)

    def drain(i, carry):
        pltpu.make_async_copy(table_hbm.at[pl.ds(0, L)], tbuf.at[pl.ds(0, L)], sem).wait()
        return carry

    lax.fori_loop(0, BPW, drain, 0)


@functools.partial(jax.jit)
def kernel(pos_enc, input_len):
    mesh = plsc.VectorSubcoreMesh(core_axis_name="c", subcore_axis_name="s")
    kfn = pl.kernel(
        _pos_body,
        out_type=jax.ShapeDtypeStruct((B * L, D), jnp.float32),
        mesh=mesh,
        scratch_types=[
            pltpu.VMEM((BPW,), jnp.int32),
            pltpu.VMEM((L + 16, D), jnp.float32),
            pltpu.VMEM((ZR, D), jnp.float32),
            pltpu.VMEM((BPW * 8, D), jnp.float32),
            pltpu.SemaphoreType.DMA,
            pltpu.SemaphoreType.DMA,
        ],
        compiler_params=pltpu.CompilerParams(needs_layout_passes=False),
    )
    out = kfn(pos_enc, input_len.astype(jnp.int32))
    return out.reshape(B, L, D)


# speedup vs baseline: 2.7855x; 1.0032x over previous
import functools

import jax
import jax.numpy as jnp
from jax import lax
from jax.experimental import pallas as pl
from jax.experimental.pallas import tpu as pltpu
from jax.experimental.pallas import tpu_sc as plsc

B = 1024
L = 512
D = 128
NC = 2
NS = 16
NW = NC * NS
BPW = B // NW
ZR = 128


def _pos_body(table_hbm, len_hbm, out_hbm, lens_v, tbuf, zbuf, bbuf, sem, dsem):
    wid = lax.axis_index("s") * NC + lax.axis_index("c")
    pltpu.sync_copy(len_hbm.at[pl.ds(wid * BPW, BPW)], lens_v)
    tld = pltpu.async_copy(table_hbm, tbuf, dsem)

    zero = jnp.zeros((16,), jnp.float32)

    def zrow(r, carry):
        for j in range(D // 16):
            zbuf[r, pl.ds(j * 16, 16)] = zero
        return carry

    lax.fori_loop(0, ZR, zrow, 0)
    tld.wait()

    lane = lax.iota(jnp.int32, 16)
    lens_lo = lens_v[pl.ds(0, 16)]
    lens_hi = lens_v[pl.ds(16, 16)]
    obase = wid * (BPW * L)

    def batch_body(b_loc, carry):
        len_s = jnp.maximum(
            jnp.max(jnp.where(lane == b_loc, lens_lo, 0)),
            jnp.max(jnp.where(lane + 16 == b_loc, lens_hi, 0)))
        row0 = obase + b_loc * L
        q8 = len_s & ~7
        r8 = len_s & 7

        off = jnp.int32(0)
        for s in (256, 128, 64, 32, 16, 8):
            bit = q8 & s

            @pl.when(bit != 0)
            def _():
                pltpu.async_copy(
                    tbuf.at[pl.ds(pl.multiple_of(off, 8), s)],
                    out_hbm.at[pl.ds(pl.multiple_of(row0 + off, 8), s)], sem)

            off = off + bit

        bb = b_loc * 8
        for j in range(8):
            keep = j < r8
            for k in range(D // 16):
                tv = tbuf[q8 + j, pl.ds(k * 16, 16)]
                bbuf[bb + j, pl.ds(k * 16, 16)] = jnp.where(keep, tv, zero)
        pltpu.async_copy(
            bbuf.at[pl.ds(bb, 8)], out_hbm.at[pl.ds(pl.multiple_of(row0 + q8, 8), 8)], sem)
        off = off + 8

        t = L - q8 - 8
        for _i in range(3):
            c = t >= ZR

            @pl.when(c)
            def _():
                pltpu.async_copy(
                    zbuf, out_hbm.at[pl.ds(pl.multiple_of(row0 + off, 8), ZR)], sem)

            dec = jnp.where(c, ZR, 0).astype(jnp.int32)
            off = off + dec
            t = t - dec
        for s in (64, 32, 16, 8):
            bit = t & s

            @pl.when(bit != 0)
            def _():
                pltpu.async_copy(
                    zbuf.at[pl.ds(0, s)],
                    out_hbm.at[pl.ds(pl.multiple_of(row0 + off, 8), s)], sem)

            off = off + bit
        return carry

    lax.fori_loop(0, BPW, batch_body, 0)

    def drain(i, carry):
        pltpu.make_async_copy(table_hbm, tbuf, sem).wait()
        return carry

    lax.fori_loop(0, BPW, drain, 0)


@functools.partial(jax.jit)
def kernel(pos_enc, input_len):
    mesh = plsc.VectorSubcoreMesh(core_axis_name="c", subcore_axis_name="s")
    kfn = pl.kernel(
        _pos_body,
        out_type=jax.ShapeDtypeStruct((B * L, D), jnp.float32),
        mesh=mesh,
        scratch_types=[
            pltpu.VMEM((BPW,), jnp.int32),
            pltpu.VMEM((L, D), jnp.float32),
            pltpu.VMEM((ZR, D), jnp.float32),
            pltpu.VMEM((BPW * 8, D), jnp.float32),
            pltpu.SemaphoreType.DMA,
            pltpu.SemaphoreType.DMA,
        ],
        compiler_params=pltpu.CompilerParams(needs_layout_passes=False),
    )
    out = kfn(pos_enc[1:L + 1], input_len.astype(jnp.int32))
    return out.reshape(B, L, D)
